# baseline (speedup 1.0000x reference)
import functools

import jax
import jax.numpy as jnp
from jax import lax
from jax.experimental import pallas as pl
from jax.experimental.pallas import tpu as pltpu
from jax.experimental.pallas import tpu_sc as plsc

N_CELLS = 1024
N_GENES = 500
N_EMB = 100
N_IDX_PAD = 512

_NC = 2
_NS = 16
BLOCK_E = 4


def _make_sc_gather():
    mesh = plsc.VectorSubcoreMesh(core_axis_name="c", subcore_axis_name="s")
    per_w = N_IDX_PAD // (_NC * _NS)

    @functools.partial(
        pl.kernel,
        mesh=mesh,
        out_type=jax.ShapeDtypeStruct((N_IDX_PAD,), jnp.float32),
        scratch_types=[
            pltpu.VMEM((per_w,), jnp.int32),
            pltpu.VMEM((per_w,), jnp.float32),
            pltpu.SemaphoreType.DMA,
        ],
    )
    def gather_bias(table_hbm, idx_hbm, out_hbm, idx_v, rows_v, sem):
        wid = lax.axis_index("s") * _NC + lax.axis_index("c")
        base = wid * per_w
        pltpu.sync_copy(idx_hbm.at[pl.ds(base, per_w)], idx_v)
        pltpu.async_copy(table_hbm.at[idx_v], rows_v, sem).wait()
        pltpu.sync_copy(rows_v, out_hbm.at[pl.ds(base, per_w)])

    return gather_bias


_sc_gather = _make_sc_gather()


def _matvec_body(x_ref, w_ref, o_ref):
    i = pl.program_id(0)
    s = x_ref[0] * w_ref[i * BLOCK_E, 0]
    for k in range(1, BLOCK_E):
        s += x_ref[k] * w_ref[i * BLOCK_E + k, 0]

    @pl.when(i == 0)
    def _init():
        o_ref[...] = s

    @pl.when(i > 0)
    def _acc():
        o_ref[...] += s


def _tc_matvec(xt, w2):
    grid = (N_EMB // BLOCK_E,)
    return pl.pallas_call(
        _matvec_body,
        grid=grid,
        in_specs=[
            pl.BlockSpec((BLOCK_E, N_GENES, N_CELLS), lambda i: (i, 0, 0)),
            pl.BlockSpec(memory_space=pltpu.SMEM),
        ],
        out_specs=pl.BlockSpec((N_GENES, N_CELLS), lambda i: (0, 0)),
        out_shape=jax.ShapeDtypeStruct((N_GENES, N_CELLS), jnp.float32),
    )(xt, w2)


def _bias_body(p_ref, b_ref, o_ref):
    o_ref[...] = p_ref[...] + b_ref[...]


def _tc_bias_add(partial_t, bias2):
    return pl.pallas_call(
        _bias_body,
        in_specs=[
            pl.BlockSpec((N_GENES, N_CELLS), lambda: (0, 0)),
            pl.BlockSpec((N_GENES, 1), lambda: (0, 0)),
        ],
        out_specs=pl.BlockSpec((N_GENES, N_CELLS), lambda: (0, 0)),
        out_shape=jax.ShapeDtypeStruct((N_GENES, N_CELLS), jnp.float32),
    )(partial_t, bias2)


def kernel(cell_gene_embedding, gene_ix, weight1, bias1):
    xt = jnp.transpose(cell_gene_embedding, (2, 1, 0))
    partial_t = _tc_matvec(xt, weight1.reshape(N_EMB, 1))
    idx = jnp.pad(gene_ix.astype(jnp.int32), (0, N_IDX_PAD - N_GENES))
    bias_g = _sc_gather(bias1, idx)[:N_GENES]
    out_t = _tc_bias_add(partial_t, bias_g.reshape(N_GENES, 1))
    return out_t.T


# speedup vs baseline: 1.1991x; 1.1991x over previous
import functools

import jax
import jax.numpy as jnp
from jax import lax
from jax.experimental import pallas as pl
from jax.experimental.pallas import tpu as pltpu
from jax.experimental.pallas import tpu_sc as plsc

N_CELLS = 1024
N_GENES = 500
N_EMB = 100
N_IDX_PAD = 512

_NC = 2
_NS = 16
BLOCK_E = 5


def _make_sc_gather():
    mesh = plsc.VectorSubcoreMesh(core_axis_name="c", subcore_axis_name="s")
    per_w = N_IDX_PAD // (_NC * _NS)

    @functools.partial(
        pl.kernel,
        mesh=mesh,
        out_type=jax.ShapeDtypeStruct((N_IDX_PAD,), jnp.float32),
        scratch_types=[
            pltpu.VMEM((per_w,), jnp.int32),
            pltpu.VMEM((per_w,), jnp.float32),
            pltpu.SemaphoreType.DMA,
        ],
    )
    def gather_bias(table_hbm, idx_hbm, out_hbm, idx_v, rows_v, sem):
        wid = lax.axis_index("s") * _NC + lax.axis_index("c")
        base = wid * per_w
        pltpu.sync_copy(idx_hbm.at[pl.ds(base, per_w)], idx_v)
        pltpu.async_copy(table_hbm.at[idx_v], rows_v, sem).wait()
        pltpu.sync_copy(rows_v, out_hbm.at[pl.ds(base, per_w)])

    return gather_bias


_sc_gather = _make_sc_gather()


def _matvec_body(x_ref, w_ref, o_ref):
    i = pl.program_id(0)
    s = x_ref[0] * w_ref[i * BLOCK_E, 0]
    for k in range(1, BLOCK_E):
        s += x_ref[k] * w_ref[i * BLOCK_E + k, 0]

    @pl.when(i == 0)
    def _init():
        o_ref[...] = s

    @pl.when(i > 0)
    def _acc():
        o_ref[...] += s


def _tc_matvec(xt, w2):
    grid = (N_EMB // BLOCK_E,)
    return pl.pallas_call(
        _matvec_body,
        grid=grid,
        in_specs=[
            pl.BlockSpec((BLOCK_E, N_GENES, N_CELLS), lambda i: (i, 0, 0)),
            pl.BlockSpec(memory_space=pltpu.SMEM),
        ],
        out_specs=pl.BlockSpec((N_GENES, N_CELLS), lambda i: (0, 0)),
        out_shape=jax.ShapeDtypeStruct((N_GENES, N_CELLS), jnp.float32),
    )(xt, w2)


def _bias_body(p_ref, b_ref, o_ref):
    o_ref[...] = p_ref[...] + b_ref[...]


def _tc_bias_add(partial_t, bias2):
    return pl.pallas_call(
        _bias_body,
        in_specs=[
            pl.BlockSpec((N_GENES, N_CELLS), lambda: (0, 0)),
            pl.BlockSpec((N_GENES, 1), lambda: (0, 0)),
        ],
        out_specs=pl.BlockSpec((N_GENES, N_CELLS), lambda: (0, 0)),
        out_shape=jax.ShapeDtypeStruct((N_GENES, N_CELLS), jnp.float32),
    )(partial_t, bias2)


def kernel(cell_gene_embedding, gene_ix, weight1, bias1):
    xt = jnp.transpose(cell_gene_embedding, (2, 1, 0))
    partial_t = _tc_matvec(xt, weight1.reshape(N_EMB, 1))
    bias_g = jnp.take(bias1, gene_ix, axis=0)
    out_t = _tc_bias_add(partial_t, bias_g.reshape(N_GENES, 1))
    return out_t.T
